# baseline (speedup 1.0000x reference)
import functools

import jax
import jax.numpy as jnp
from jax import lax
from jax.experimental import pallas as pl
from jax.experimental.pallas import tpu as pltpu
from jax.experimental.pallas import tpu_sc as plsc

_B, _K, _N, _R = 16384, 4, 100000, 8
_RR = _R * _R
_NC, _NS = 2, 16
_NW = _NC * _NS
_BPW = _B // _NW
_CHI = 128
_NCH = _BPW // _CHI
_CH = 8704
_CH2 = _CH // 2


def _softplus(x):
    return jnp.maximum(x, 0.0) + jnp.log1p(jnp.exp(-jnp.abs(x)))


def _norm_repack_call(lct):
    n = lct.shape[1]
    ch = _CH
    grid = (n + ch - 1) // ch

    def body(x_ref, o_ref, tab_ref, acc_ref):
        g = pl.program_id(0)

        @pl.when(g == 0)
        def _init():
            acc_ref[...] = jnp.zeros_like(acc_ref)

        x = x_ref[...]
        col = g * ch + lax.broadcasted_iota(jnp.int32, (_K * _RR, ch), 1)
        acc_ref[...] += jnp.where(col < n, _softplus(x), 0.0)

        xt = jnp.swapaxes(x, 0, 1)
        ev = xt[: ch // 2, :]
        od = xt[ch // 2 :, :]
        tabs = [
            jnp.concatenate(
                [
                    ev[:, kk * _RR : (kk + 1) * _RR],
                    od[:, kk * _RR : (kk + 1) * _RR],
                ],
                axis=1,
            )
            for kk in range(_K)
        ]
        tab_ref[...] = jnp.stack(tabs, axis=0)

        @pl.when(g == grid - 1)
        def _fin():
            s = jnp.sum(acc_ref[...], axis=1)
            m = s.reshape(_K, _R, _R)
            t_ = m[0]
            for i in range(1, _K):
                t_ = sum(
                    t_[:, u : u + 1] * m[i][u : u + 1, :] for u in range(_R)
                )
            eye = lax.broadcasted_iota(jnp.int32, (_R, _R), 0) == \
                lax.broadcasted_iota(jnp.int32, (_R, _R), 1)
            o_ref[0, 0] = jnp.log(jnp.sum(jnp.where(eye, t_, 0.0)))

    return pl.pallas_call(
        body,
        grid=(grid,),
        in_specs=[pl.BlockSpec((_K * _RR, ch), lambda g: (0, g))],
        out_specs=[
            pl.BlockSpec(memory_space=pltpu.SMEM),
            pl.BlockSpec((_K, ch // 2, 128), lambda g: (0, g, 0)),
        ],
        out_shape=[
            jax.ShapeDtypeStruct((1, 1), jnp.float32),
            jax.ShapeDtypeStruct((_K, grid * (ch // 2), 128), jnp.float32),
        ],
        scratch_shapes=[pltpu.VMEM((_K * _RR, ch), jnp.float32)],
    )(lct)


def _sc_gather(tab, fidx4):
    mesh = plsc.VectorSubcoreMesh(core_axis_name="c", subcore_axis_name="s")

    @functools.partial(
        pl.kernel,
        mesh=mesh,
        compiler_params=pltpu.CompilerParams(use_tc_tiling_on_sc=False),
        out_type=jax.ShapeDtypeStruct((_K, _B, 128), jnp.float32),
        scratch_types=[
            pltpu.VMEM((_NCH, _CHI), jnp.int32),
            pltpu.VMEM((_BPW, 128), jnp.float32),
            pltpu.SemaphoreType.DMA,
            pltpu.SemaphoreType.DMA,
        ],
    )
    def body(tab_hbm, fidx_hbm, out_hbm, idx_v, rows_v, gsem, ssem):
        wid = lax.axis_index("s") * _NC + lax.axis_index("c")
        base = wid * _BPW
        for kk in range(_K):
            pltpu.sync_copy(fidx_hbm.at[kk, wid], idx_v)
            copies = [
                pltpu.async_copy(
                    tab_hbm.at[kk].at[idx_v.at[j]],
                    rows_v.at[pl.ds(j * _CHI, _CHI)],
                    gsem,
                )
                for j in range(_NCH)
            ]
            for c in copies:
                c.wait()
            wr = pltpu.async_copy(
                rows_v, out_hbm.at[kk, pl.ds(base, _BPW)], ssem
            )
            wr.wait()

    return body(tab, fidx4)


def _chain_call(g2, par, ln):
    bb = 4096
    grid = _B // bb

    def body(x_ref, p_ref, ln_ref, o_ref):
        x = x_ref[...]
        m = []
        for kk in range(_K):
            xt = jnp.swapaxes(x[kk], 0, 1)
            selk = p_ref[kk : kk + 1, :] == 1
            xk = jnp.where(selk, xt[_RR:, :], xt[:_RR, :])
            m.append(_softplus(xk).reshape(_R, _R, bb))
        p = m[0]
        for kk in range(1, _K):
            p = sum(
                p[:, u : u + 1, :] * m[kk][u : u + 1, :, :]
                for u in range(_R)
            )
        eye = (
            lax.broadcasted_iota(jnp.int32, (_R, _R, 1), 0)
            == lax.broadcasted_iota(jnp.int32, (_R, _R, 1), 1)
        )
        t_ = jnp.sum(jnp.where(eye, p, 0.0), axis=(0, 1))
        o_ref[...] = (jnp.log(t_) - ln_ref[0, 0]).reshape(1, bb)

    return pl.pallas_call(
        body,
        grid=(grid,),
        in_specs=[
            pl.BlockSpec((_K, bb, 128), lambda g: (0, g, 0)),
            pl.BlockSpec((_K, bb), lambda g: (0, g)),
            pl.BlockSpec(memory_space=pltpu.SMEM),
        ],
        out_specs=pl.BlockSpec((1, bb), lambda g: (0, g)),
        out_shape=jax.ShapeDtypeStruct((1, _B), jnp.float32),
    )(g2, par, ln)


def kernel(index, log_cores):
    k, n, r = log_cores.shape[0], log_cores.shape[1], log_cores.shape[2]
    b = index.shape[0]
    lct = jnp.transpose(log_cores, (0, 2, 3, 1)).reshape(k * r * r, n)
    it = index.T
    off = it % _CH
    half = off // _CH2
    row = (it // _CH) * _CH2 + off - half * _CH2
    fidx4 = row.reshape(k, _NW, _NCH, _CHI)
    par = half.astype(jnp.int32)
    ln, tab = _norm_repack_call(lct)
    g2 = _sc_gather(tab, fidx4)
    out2 = _chain_call(g2, par, ln)
    return out2.reshape(b)


# speedup vs baseline: 1.0217x; 1.0217x over previous
import functools

import jax
import jax.numpy as jnp
from jax import lax
from jax.experimental import pallas as pl
from jax.experimental.pallas import tpu as pltpu
from jax.experimental.pallas import tpu_sc as plsc

_B, _K, _N, _R = 16384, 4, 100000, 8
_RR = _R * _R
_NC, _NS = 2, 16
_NW = _NC * _NS
_BPW = _B // _NW
_CHI = 128
_NCH = _BPW // _CHI
_CH = 4352
_CH2 = _CH // 2


def _softplus(x):
    return jnp.maximum(x, 0.0) + jnp.log1p(jnp.exp(-jnp.abs(x)))


def _norm_repack_call(lct):
    n = lct.shape[1]
    ch = _CH
    grid = (n + ch - 1) // ch

    def body(x_ref, o_ref, tab_ref, acc_ref):
        g = pl.program_id(0)

        @pl.when(g == 0)
        def _init():
            acc_ref[...] = jnp.zeros_like(acc_ref)

        x = x_ref[...]
        col = g * ch + lax.broadcasted_iota(jnp.int32, (_K * _RR, ch), 1)
        acc_ref[...] += jnp.where(col < n, _softplus(x), 0.0)

        xt = jnp.swapaxes(x, 0, 1)
        ev = xt[: ch // 2, :]
        od = xt[ch // 2 :, :]
        tabs = [
            jnp.concatenate(
                [
                    ev[:, kk * _RR : (kk + 1) * _RR],
                    od[:, kk * _RR : (kk + 1) * _RR],
                ],
                axis=1,
            )
            for kk in range(_K)
        ]
        tab_ref[...] = jnp.stack(tabs, axis=0)

        @pl.when(g == grid - 1)
        def _fin():
            s = jnp.sum(acc_ref[...], axis=1)
            m = s.reshape(_K, _R, _R)
            t_ = m[0]
            for i in range(1, _K):
                t_ = sum(
                    t_[:, u : u + 1] * m[i][u : u + 1, :] for u in range(_R)
                )
            eye = lax.broadcasted_iota(jnp.int32, (_R, _R), 0) == \
                lax.broadcasted_iota(jnp.int32, (_R, _R), 1)
            o_ref[0, 0] = jnp.log(jnp.sum(jnp.where(eye, t_, 0.0)))

    return pl.pallas_call(
        body,
        grid=(grid,),
        in_specs=[pl.BlockSpec((_K * _RR, ch), lambda g: (0, g))],
        out_specs=[
            pl.BlockSpec(memory_space=pltpu.SMEM),
            pl.BlockSpec((_K, ch // 2, 128), lambda g: (0, g, 0)),
        ],
        out_shape=[
            jax.ShapeDtypeStruct((1, 1), jnp.float32),
            jax.ShapeDtypeStruct((_K, grid * (ch // 2), 128), jnp.float32),
        ],
        scratch_shapes=[pltpu.VMEM((_K * _RR, ch), jnp.float32)],
    )(lct)


def _sc_gather(tab, fidx4):
    mesh = plsc.VectorSubcoreMesh(core_axis_name="c", subcore_axis_name="s")

    @functools.partial(
        pl.kernel,
        mesh=mesh,
        compiler_params=pltpu.CompilerParams(use_tc_tiling_on_sc=False),
        out_type=jax.ShapeDtypeStruct((_K, _B, 128), jnp.float32),
        scratch_types=[
            pltpu.VMEM((_NCH, _CHI), jnp.int32),
            pltpu.VMEM((_BPW, 128), jnp.float32),
            pltpu.SemaphoreType.DMA,
            pltpu.SemaphoreType.DMA,
        ],
    )
    def body(tab_hbm, fidx_hbm, out_hbm, idx_v, rows_v, gsem, ssem):
        wid = lax.axis_index("s") * _NC + lax.axis_index("c")
        base = wid * _BPW
        for kk in range(_K):
            pltpu.sync_copy(fidx_hbm.at[kk, wid], idx_v)
            copies = [
                pltpu.async_copy(
                    tab_hbm.at[kk].at[idx_v.at[j]],
                    rows_v.at[pl.ds(j * _CHI, _CHI)],
                    gsem,
                )
                for j in range(_NCH)
            ]
            for c in copies:
                c.wait()
            wr = pltpu.async_copy(
                rows_v, out_hbm.at[kk, pl.ds(base, _BPW)], ssem
            )
            wr.wait()

    return body(tab, fidx4)


def _chain_call(g2, par, ln):
    bb = 4096
    grid = _B // bb

    def body(x_ref, p_ref, ln_ref, o_ref):
        x = x_ref[...]
        m = []
        for kk in range(_K):
            xt = jnp.swapaxes(x[kk], 0, 1)
            selk = p_ref[kk : kk + 1, :] == 1
            xk = jnp.where(selk, xt[_RR:, :], xt[:_RR, :])
            m.append(_softplus(xk).reshape(_R, _R, bb))
        p = m[0]
        for kk in range(1, _K):
            p = sum(
                p[:, u : u + 1, :] * m[kk][u : u + 1, :, :]
                for u in range(_R)
            )
        eye = (
            lax.broadcasted_iota(jnp.int32, (_R, _R, 1), 0)
            == lax.broadcasted_iota(jnp.int32, (_R, _R, 1), 1)
        )
        t_ = jnp.sum(jnp.where(eye, p, 0.0), axis=(0, 1))
        o_ref[...] = (jnp.log(t_) - ln_ref[0, 0]).reshape(1, bb)

    return pl.pallas_call(
        body,
        grid=(grid,),
        in_specs=[
            pl.BlockSpec((_K, bb, 128), lambda g: (0, g, 0)),
            pl.BlockSpec((_K, bb), lambda g: (0, g)),
            pl.BlockSpec(memory_space=pltpu.SMEM),
        ],
        out_specs=pl.BlockSpec((1, bb), lambda g: (0, g)),
        out_shape=jax.ShapeDtypeStruct((1, _B), jnp.float32),
    )(g2, par, ln)


def kernel(index, log_cores):
    k, n, r = log_cores.shape[0], log_cores.shape[1], log_cores.shape[2]
    b = index.shape[0]
    lct = jnp.transpose(log_cores, (0, 2, 3, 1)).reshape(k * r * r, n)
    it = index.T
    off = it % _CH
    half = off // _CH2
    row = (it // _CH) * _CH2 + off - half * _CH2
    fidx4 = row.reshape(k, _NW, _NCH, _CHI)
    par = half.astype(jnp.int32)
    ln, tab = _norm_repack_call(lct)
    g2 = _sc_gather(tab, fidx4)
    out2 = _chain_call(g2, par, ln)
    return out2.reshape(b)


# speedup vs baseline: 1.0261x; 1.0043x over previous
import functools

import jax
import jax.numpy as jnp
from jax import lax
from jax.experimental import pallas as pl
from jax.experimental.pallas import tpu as pltpu
from jax.experimental.pallas import tpu_sc as plsc

_B, _K, _N, _R = 16384, 4, 100000, 8
_RR = _R * _R
_NC, _NS = 2, 16
_NW = _NC * _NS
_BPW = _B // _NW
_CHI = 128
_NCH = _BPW // _CHI
_CH = 4352
_CH2 = _CH // 2


def _softplus(x):
    return jnp.maximum(x, 0.0) + jnp.log1p(jnp.exp(-jnp.abs(x)))


def _norm_repack_call(lct):
    n = lct.shape[1]
    ch = _CH
    grid = (n + ch - 1) // ch

    def body(x_ref, o_ref, tab_ref, acc_ref):
        g = pl.program_id(0)

        @pl.when(g == 0)
        def _init():
            acc_ref[...] = jnp.zeros_like(acc_ref)

        x = x_ref[...]
        col = g * ch + lax.broadcasted_iota(jnp.int32, (_K * _RR, ch), 1)
        acc_ref[...] += jnp.where(col < n, _softplus(x), 0.0)

        xt = jnp.swapaxes(x, 0, 1)
        ev = xt[: ch // 2, :]
        od = xt[ch // 2 :, :]
        tabs = [
            jnp.concatenate(
                [
                    ev[:, kk * _RR : (kk + 1) * _RR],
                    od[:, kk * _RR : (kk + 1) * _RR],
                ],
                axis=1,
            )
            for kk in range(_K)
        ]
        tab_ref[...] = jnp.stack(tabs, axis=0)

        @pl.when(g == grid - 1)
        def _fin():
            s = jnp.sum(acc_ref[...], axis=1)
            m = s.reshape(_K, _R, _R)
            t_ = m[0]
            for i in range(1, _K):
                t_ = sum(
                    t_[:, u : u + 1] * m[i][u : u + 1, :] for u in range(_R)
                )
            eye = lax.broadcasted_iota(jnp.int32, (_R, _R), 0) == \
                lax.broadcasted_iota(jnp.int32, (_R, _R), 1)
            o_ref[0, 0] = jnp.log(jnp.sum(jnp.where(eye, t_, 0.0)))

    return pl.pallas_call(
        body,
        grid=(grid,),
        in_specs=[pl.BlockSpec((_K * _RR, ch), lambda g: (0, g))],
        out_specs=[
            pl.BlockSpec(memory_space=pltpu.SMEM),
            pl.BlockSpec((_K, ch // 2, 128), lambda g: (0, g, 0)),
        ],
        out_shape=[
            jax.ShapeDtypeStruct((1, 1), jnp.float32),
            jax.ShapeDtypeStruct((_K, grid * (ch // 2), 128), jnp.float32),
        ],
        scratch_shapes=[pltpu.VMEM((_K * _RR, ch), jnp.float32)],
    )(lct)


def _sc_gather(tab, fidx4):
    mesh = plsc.VectorSubcoreMesh(core_axis_name="c", subcore_axis_name="s")

    @functools.partial(
        pl.kernel,
        mesh=mesh,
        compiler_params=pltpu.CompilerParams(use_tc_tiling_on_sc=False),
        out_type=jax.ShapeDtypeStruct((_K, _B, 128), jnp.float32),
        scratch_types=[
            pltpu.VMEM((_NCH, _CHI), jnp.int32),
            pltpu.VMEM((_BPW, 128), jnp.float32),
            pltpu.SemaphoreType.DMA,
            pltpu.SemaphoreType.DMA,
        ],
    )
    def body(tab_hbm, fidx_hbm, out_hbm, idx_v, rows_v, gsem, ssem):
        wid = lax.axis_index("s") * _NC + lax.axis_index("c")
        base = wid * _BPW
        for kk in range(_K):
            pltpu.sync_copy(fidx_hbm.at[kk, wid], idx_v)
            copies = [
                pltpu.async_copy(
                    tab_hbm.at[kk].at[idx_v.at[j]],
                    rows_v.at[pl.ds(j * _CHI, _CHI)],
                    gsem,
                )
                for j in range(_NCH)
            ]
            for c in copies:
                c.wait()
            wr = pltpu.async_copy(
                rows_v, out_hbm.at[kk, pl.ds(base, _BPW)], ssem
            )
            wr.wait()

    return body(tab, fidx4)


def _chain_call(g2, par, ln):
    bb = 2048
    grid = _B // bb

    def body(x_ref, p_ref, ln_ref, o_ref):
        x = x_ref[...]
        m = []
        for kk in range(_K):
            xt = jnp.swapaxes(x[kk], 0, 1)
            selk = p_ref[kk : kk + 1, :] == 1
            xk = jnp.where(selk, xt[_RR:, :], xt[:_RR, :])
            m.append(_softplus(xk).reshape(_R, _R, bb))
        p = m[0]
        for kk in range(1, _K):
            p = sum(
                p[:, u : u + 1, :] * m[kk][u : u + 1, :, :]
                for u in range(_R)
            )
        eye = (
            lax.broadcasted_iota(jnp.int32, (_R, _R, 1), 0)
            == lax.broadcasted_iota(jnp.int32, (_R, _R, 1), 1)
        )
        t_ = jnp.sum(jnp.where(eye, p, 0.0), axis=(0, 1))
        o_ref[...] = (jnp.log(t_) - ln_ref[0, 0]).reshape(1, bb)

    return pl.pallas_call(
        body,
        grid=(grid,),
        in_specs=[
            pl.BlockSpec((_K, bb, 128), lambda g: (0, g, 0)),
            pl.BlockSpec((_K, bb), lambda g: (0, g)),
            pl.BlockSpec(memory_space=pltpu.SMEM),
        ],
        out_specs=pl.BlockSpec((1, bb), lambda g: (0, g)),
        out_shape=jax.ShapeDtypeStruct((1, _B), jnp.float32),
    )(g2, par, ln)


def kernel(index, log_cores):
    k, n, r = log_cores.shape[0], log_cores.shape[1], log_cores.shape[2]
    b = index.shape[0]
    lct = jnp.transpose(log_cores, (0, 2, 3, 1)).reshape(k * r * r, n)
    it = index.T
    off = it % _CH
    half = off // _CH2
    row = (it // _CH) * _CH2 + off - half * _CH2
    fidx4 = row.reshape(k, _NW, _NCH, _CHI)
    par = half.astype(jnp.int32)
    ln, tab = _norm_repack_call(lct)
    g2 = _sc_gather(tab, fidx4)
    out2 = _chain_call(g2, par, ln)
    return out2.reshape(b)
